# Initial kernel scaffold; baseline (speedup 1.0000x reference)
import jax
import jax.numpy as jnp
from jax.experimental import pallas as pl


def kernel(x, emb_table):
    raise NotImplementedError("write your pallas kernel here")


# speedup vs baseline: 3.9071x; 3.9071x over previous
import functools

import jax
import jax.numpy as jnp
from jax.experimental import pallas as pl

D_MODEL = 1024
HALF = D_MODEL // 2
C_ROWS = 16


def _build_v_kernel(emb_ref, v_ref):
    e = emb_ref[...]
    norms = jnp.sqrt(jnp.sum(e * e, axis=1, keepdims=True))
    scale = jnp.where(norms > 1.0, 1.0 / (norms + 1e-7), 1.0)
    e = e * scale
    row = jax.lax.broadcasted_iota(jnp.int32, (HALF, D_MODEL), 0)
    col = jax.lax.broadcasted_iota(jnp.int32, (HALF, D_MODEL), 1)
    p = (col == 2 * row).astype(jnp.float32)
    v_ref[...] = jax.lax.dot(e, p, precision=jax.lax.Precision.HIGHEST)


def _add_kernel(x_ref, v_ref, o_ref):
    o_ref[...] = x_ref[...] + v_ref[...]


@jax.jit
def kernel(x, emb_table):
    B, C, S, D = x.shape

    v = pl.pallas_call(
        _build_v_kernel,
        out_shape=jax.ShapeDtypeStruct((C_ROWS, D_MODEL), jnp.float32),
        grid=(1,),
        in_specs=[pl.BlockSpec((C_ROWS, HALF), lambda i: (0, 0))],
        out_specs=pl.BlockSpec((C_ROWS, D_MODEL), lambda i: (0, 0)),
    )(emb_table)

    xr = x.reshape(B * C, S, D)
    v3 = v.reshape(C_ROWS, 1, D_MODEL)
    s_blk = 512
    out = pl.pallas_call(
        _add_kernel,
        out_shape=jax.ShapeDtypeStruct(xr.shape, xr.dtype),
        grid=(B * C, S // s_blk),
        in_specs=[
            pl.BlockSpec((1, s_blk, D), lambda i, j: (i, j, 0)),
            pl.BlockSpec((1, 1, D), lambda i, j: (C_ROWS - 1 - (i % C_ROWS), 0, 0)),
        ],
        out_specs=pl.BlockSpec((1, s_blk, D), lambda i, j: (i, j, 0)),
    )(xr, v3)
    return out.reshape(B, C, S, D)


# speedup vs baseline: 4.2435x; 1.0861x over previous
import functools

import jax
import jax.numpy as jnp
from jax.experimental import pallas as pl

D_MODEL = 1024
HALF = D_MODEL // 2
C_ROWS = 16


def _build_v_kernel(emb_ref, v_ref):
    e = emb_ref[...]
    norms = jnp.sqrt(jnp.sum(e * e, axis=1, keepdims=True))
    scale = jnp.where(norms > 1.0, 1.0 / (norms + 1e-7), 1.0)
    e = e * scale
    row = jax.lax.broadcasted_iota(jnp.int32, (HALF, D_MODEL), 0)
    col = jax.lax.broadcasted_iota(jnp.int32, (HALF, D_MODEL), 1)
    p = (col == 2 * row).astype(jnp.float32)
    v_ref[...] = jax.lax.dot(e, p, precision=jax.lax.Precision.HIGHEST)


def _add_kernel(x_ref, v_ref, o_ref):
    o_ref[...] = x_ref[...] + v_ref[...]


@jax.jit
def kernel(x, emb_table):
    B, C, S, D = x.shape

    v = pl.pallas_call(
        _build_v_kernel,
        out_shape=jax.ShapeDtypeStruct((C_ROWS, D_MODEL), jnp.float32),
        grid=(1,),
        in_specs=[pl.BlockSpec((C_ROWS, HALF), lambda i: (0, 0))],
        out_specs=pl.BlockSpec((C_ROWS, D_MODEL), lambda i: (0, 0)),
    )(emb_table)

    xr = x.reshape(B * C, S, D)
    v3 = v.reshape(C_ROWS, 1, D_MODEL)
    s_blk = 1024
    out = pl.pallas_call(
        _add_kernel,
        out_shape=jax.ShapeDtypeStruct(xr.shape, xr.dtype),
        grid=(B * C, S // s_blk),
        in_specs=[
            pl.BlockSpec((1, s_blk, D), lambda i, j: (i, j, 0)),
            pl.BlockSpec((1, 1, D), lambda i, j: (C_ROWS - 1 - (i % C_ROWS), 0, 0)),
        ],
        out_specs=pl.BlockSpec((1, s_blk, D), lambda i, j: (i, j, 0)),
    )(xr, v3)
    return out.reshape(B, C, S, D)


# speedup vs baseline: 4.2971x; 1.0126x over previous
import functools

import jax
import jax.numpy as jnp
from jax.experimental import pallas as pl

D_MODEL = 1024
HALF = D_MODEL // 2
C_ROWS = 16


def _build_v_kernel(emb_ref, v_ref):
    e = emb_ref[...]
    norms = jnp.sqrt(jnp.sum(e * e, axis=1, keepdims=True))
    scale = jnp.where(norms > 1.0, 1.0 / (norms + 1e-7), 1.0)
    e = e * scale
    row = jax.lax.broadcasted_iota(jnp.int32, (HALF, D_MODEL), 0)
    col = jax.lax.broadcasted_iota(jnp.int32, (HALF, D_MODEL), 1)
    p = (col == 2 * row).astype(jnp.float32)
    v_ref[...] = jax.lax.dot(e, p, precision=jax.lax.Precision.HIGHEST)


def _add_kernel(x_ref, v_ref, o_ref):
    o_ref[...] = x_ref[...] + v_ref[...]


@jax.jit
def kernel(x, emb_table):
    B, C, S, D = x.shape

    v = pl.pallas_call(
        _build_v_kernel,
        out_shape=jax.ShapeDtypeStruct((C_ROWS, D_MODEL), jnp.float32),
        grid=(1,),
        in_specs=[pl.BlockSpec((C_ROWS, HALF), lambda i: (0, 0))],
        out_specs=pl.BlockSpec((C_ROWS, D_MODEL), lambda i: (0, 0)),
    )(emb_table)

    xr = x.reshape(B * C, S, D)
    v3 = v.reshape(C_ROWS, 1, D_MODEL)
    s_blk = 2048
    out = pl.pallas_call(
        _add_kernel,
        out_shape=jax.ShapeDtypeStruct(xr.shape, xr.dtype),
        grid=(B * C, S // s_blk),
        in_specs=[
            pl.BlockSpec((1, s_blk, D), lambda i, j: (i, j, 0)),
            pl.BlockSpec((1, 1, D), lambda i, j: (C_ROWS - 1 - (i % C_ROWS), 0, 0)),
        ],
        out_specs=pl.BlockSpec((1, s_blk, D), lambda i, j: (i, j, 0)),
    )(xr, v3)
    return out.reshape(B, C, S, D)


# speedup vs baseline: 4.3723x; 1.0175x over previous
import functools

import jax
import jax.numpy as jnp
from jax.experimental import pallas as pl
from jax.experimental.pallas import tpu as pltpu

D_MODEL = 1024
HALF = D_MODEL // 2
C_ROWS = 16


def _fused_kernel(emb_ref, x_ref, o_ref, v_ref):
    i = pl.program_id(0)

    @pl.when(i == 0)
    def _build_v():
        e = emb_ref[...]
        norms = jnp.sqrt(jnp.sum(e * e, axis=1, keepdims=True))
        scale = jnp.where(norms > 1.0, 1.0 / (norms + 1e-7), 1.0)
        e = e * scale
        row = jax.lax.broadcasted_iota(jnp.int32, (HALF, D_MODEL), 0)
        col = jax.lax.broadcasted_iota(jnp.int32, (HALF, D_MODEL), 1)
        p = (col == 2 * row).astype(jnp.float32)
        v_ref[...] = jax.lax.dot(e, p, precision=jax.lax.Precision.HIGHEST)

    c = C_ROWS - 1 - jax.lax.rem(i, C_ROWS)
    o_ref[...] = x_ref[...] + v_ref[pl.ds(c, 1), :][None, :, :]


@jax.jit
def kernel(x, emb_table):
    B, C, S, D = x.shape
    xr = x.reshape(B * C, S, D)
    s_blk = 2048

    out = pl.pallas_call(
        _fused_kernel,
        out_shape=jax.ShapeDtypeStruct(xr.shape, xr.dtype),
        grid=(B * C, S // s_blk),
        in_specs=[
            pl.BlockSpec((C_ROWS, HALF), lambda i, j: (0, 0)),
            pl.BlockSpec((1, s_blk, D), lambda i, j: (i, j, 0)),
        ],
        out_specs=pl.BlockSpec((1, s_blk, D), lambda i, j: (i, j, 0)),
        scratch_shapes=[pltpu.VMEM((C_ROWS, D_MODEL), jnp.float32)],
    )(emb_table, xr)
    return out.reshape(B, C, S, D)
